# Initial kernel scaffold; baseline (speedup 1.0000x reference)
import jax
import jax.numpy as jnp
from jax.experimental import pallas as pl


def kernel(input_ids, codebooks, codes):
    raise NotImplementedError("write your pallas kernel here")


# speedup vs baseline: 6.5803x; 6.5803x over previous
import functools

import jax
import jax.numpy as jnp
from jax import lax
from jax.experimental import pallas as pl
from jax.experimental.pallas import tpu as pltpu
from jax.experimental.pallas import tpu_sc as plsc

_B = 4096
_L = 50
_NTOK = _B * _L
_S = 8
_CBS = 256
_D = 16
_NW = 32
_TPW = _NTOK // _NW
_T = 256
_NCH = _TPW // _T
_G = 128

_mesh = plsc.VectorSubcoreMesh(core_axis_name="c", subcore_axis_name="s")


@functools.partial(
    pl.kernel,
    out_type=jax.ShapeDtypeStruct((_NTOK * _S, _D), jnp.float32),
    mesh=_mesh,
    scratch_types=[
        pltpu.VMEM((_T // _G, _G), jnp.int32),
        pltpu.VMEM((_T, _S), jnp.int32),
        pltpu.VMEM((_T * _S // _G, _G), jnp.int32),
        pltpu.VMEM((_T * _S, _D), jnp.float32),
        pltpu.SemaphoreType.DMA,
        pltpu.SemaphoreType.DMA,
    ],
    compiler_params=pltpu.CompilerParams(use_tc_tiling_on_sc=False,
                                         needs_layout_passes=False),
)
def _pq_lookup(ids_hbm, cb_hbm, codes_hbm, out_hbm,
               ids_v, sel_v, fidx_v, rows_v, sem_a, sem_b):
    cid = lax.axis_index("c")
    sid = lax.axis_index("s")
    wid = sid * 2 + cid
    base = wid * _TPW

    lane = lax.iota(jnp.int32, 16)
    tok_half = lane >> 3
    sub = lane & (_S - 1)
    sub_off = sub * _CBS

    def chunk(c, carry):
        tok0 = base + c * _T
        for q in range(_T // _G):
            pltpu.sync_copy(ids_hbm.at[pl.ds(tok0 + q * _G, _G)], ids_v.at[q])
        for q in range(_T // _G):
            pltpu.async_copy(codes_hbm.at[ids_v.at[q]],
                             sel_v.at[pl.ds(q * _G, _G)], sem_a).wait()
        for j in range(_T * _S // _G):
            for q in range(_G // 16):
                pair = j * (_G // 16) + q
                vals = plsc.load_gather(sel_v, [2 * pair + tok_half, sub])
                fidx_v[j, pl.ds(q * 16, 16)] = vals + sub_off
            pltpu.async_copy(cb_hbm.at[fidx_v.at[j]],
                             rows_v.at[pl.ds(j * _G, _G)], sem_b).wait()
        pltpu.sync_copy(rows_v, out_hbm.at[pl.ds(tok0 * _S, _T * _S)])
        return carry

    lax.fori_loop(0, _NCH, chunk, 0)


def kernel(input_ids, codebooks, codes):
    ids2d = input_ids.reshape(_NTOK).astype(jnp.int32)
    cb2d = codebooks.reshape(_S * _CBS, _D)
    out = _pq_lookup(ids2d, cb2d, codes)
    return out.reshape(_B, _L, _S * _D)


# speedup vs baseline: 8.6480x; 1.3142x over previous
import functools

import jax
import jax.numpy as jnp
from jax import lax
from jax.experimental import pallas as pl
from jax.experimental.pallas import tpu as pltpu
from jax.experimental.pallas import tpu_sc as plsc

_B = 4096
_L = 50
_NTOK = _B * _L
_S = 8
_CBS = 256
_D = 16
_NW = 32
_TPW = _NTOK // _NW
_T = 256
_NCH = _TPW // _T
_G = 128

_mesh = plsc.VectorSubcoreMesh(core_axis_name="c", subcore_axis_name="s")


@functools.partial(
    pl.kernel,
    out_type=jax.ShapeDtypeStruct((_NTOK * _S, _D), jnp.float32),
    mesh=_mesh,
    scratch_types=[
        pltpu.VMEM((_T // _G, _G), jnp.int32),
        pltpu.VMEM((_T, _S), jnp.int32),
        pltpu.VMEM((_T * _S // _G, _G), jnp.int32),
        pltpu.VMEM((_T * _S, _D), jnp.float32),
        pltpu.SemaphoreType.DMA,
        pltpu.SemaphoreType.DMA,
    ],
    compiler_params=pltpu.CompilerParams(use_tc_tiling_on_sc=False,
                                         needs_layout_passes=False),
)
def _pq_lookup(ids_hbm, cb_hbm, codes_hbm, out_hbm,
               ids_v, sel_v, fidx_v, rows_v, sem_a, sem_b):
    cid = lax.axis_index("c")
    sid = lax.axis_index("s")
    wid = sid * 2 + cid
    base = wid * _TPW

    lane = lax.iota(jnp.int32, 16)
    tok_half = lane >> 3
    sub = lane & (_S - 1)
    sub_off = sub * _CBS

    def chunk(c, carry):
        tok0 = base + c * _T
        for q in range(_T // _G):
            pltpu.sync_copy(ids_hbm.at[pl.ds(tok0 + q * _G, _G)], ids_v.at[q])
        cdescs = [
            pltpu.async_copy(codes_hbm.at[ids_v.at[q]],
                             sel_v.at[pl.ds(q * _G, _G)], sem_a)
            for q in range(_T // _G)
        ]
        for dsc in cdescs:
            dsc.wait()
        gdescs = []
        for j in range(_T * _S // _G):
            for q in range(_G // 16):
                pair = j * (_G // 16) + q
                vals = plsc.load_gather(sel_v, [2 * pair + tok_half, sub])
                fidx_v[j, pl.ds(q * 16, 16)] = vals + sub_off
            gdescs.append(
                pltpu.async_copy(cb_hbm.at[fidx_v.at[j]],
                                 rows_v.at[pl.ds(j * _G, _G)], sem_b))
        for dsc in gdescs:
            dsc.wait()
        pltpu.sync_copy(rows_v, out_hbm.at[pl.ds(tok0 * _S, _T * _S)])
        return carry

    lax.fori_loop(0, _NCH, chunk, 0)


def kernel(input_ids, codebooks, codes):
    ids2d = input_ids.reshape(_NTOK).astype(jnp.int32)
    cb2d = codebooks.reshape(_S * _CBS, _D)
    out = _pq_lookup(ids2d, cb2d, codes)
    return out.reshape(_B, _L, _S * _D)


# speedup vs baseline: 10.0117x; 1.1577x over previous
import functools

import jax
import jax.numpy as jnp
from jax import lax
from jax.experimental import pallas as pl
from jax.experimental.pallas import tpu as pltpu
from jax.experimental.pallas import tpu_sc as plsc

_B = 4096
_L = 50
_NTOK = _B * _L
_S = 8
_CBS = 256
_D = 16
_NW = 32
_TPW = _NTOK // _NW
_T = 256
_NCH = _TPW // _T
_G = 128

_mesh = plsc.VectorSubcoreMesh(core_axis_name="c", subcore_axis_name="s")


@functools.partial(
    pl.kernel,
    out_type=jax.ShapeDtypeStruct((_NTOK * _S, _D), jnp.float32),
    mesh=_mesh,
    scratch_types=[
        pltpu.VMEM((_T // _G, _G), jnp.int32),
        pltpu.VMEM((_T // _G, _G), jnp.int32),
        pltpu.VMEM((_T, _S), jnp.int32),
        pltpu.VMEM((_T, _S), jnp.int32),
        pltpu.VMEM((_T * _S // _G, _G), jnp.int32),
        pltpu.VMEM((_T * _S // _G, _G), jnp.int32),
        pltpu.VMEM((_T * _S, _D), jnp.float32),
        pltpu.VMEM((_T * _S, _D), jnp.float32),
        pltpu.VMEM_SHARED((_S * _CBS, _D), jnp.float32),
        pltpu.SemaphoreType.DMA,
        pltpu.SemaphoreType.DMA,
        pltpu.SemaphoreType.DMA,
        pltpu.SemaphoreType.DMA,
        pltpu.SemaphoreType.DMA,
    ],
    compiler_params=pltpu.CompilerParams(use_tc_tiling_on_sc=False,
                                         needs_layout_passes=False),
)
def _pq_lookup(ids_hbm, cb_hbm, codes_hbm, out_hbm,
               ids0, ids1, sel0, sel1, fidx0, fidx1, rows0, rows1,
               cb_sp, sem_c0, sem_c1, sem_r, sem_o0, sem_o1):
    cid = lax.axis_index("c")
    sid = lax.axis_index("s")
    wid = sid * 2 + cid
    base = wid * _TPW

    _ids = (ids0, ids1)
    _sel = (sel0, sel1)
    _fidx = (fidx0, fidx1)
    _rows = (rows0, rows1)
    _sem_c = (sem_c0, sem_c1)
    _sem_o = (sem_o0, sem_o1)

    lane = lax.iota(jnp.int32, 16)
    tok_half = lane >> 3
    sub = lane & (_S - 1)
    sub_off = sub * _CBS

    def out_slice(g):
        return out_hbm.at[pl.ds((base + g * _T) * _S, _T * _S)]

    def issue_stage1(g, b):
        tok0 = base + g * _T
        for q in range(_T // _G):
            pltpu.sync_copy(ids_hbm.at[pl.ds(tok0 + q * _G, _G)],
                            _ids[b].at[q])
            pltpu.async_copy(codes_hbm.at[_ids[b].at[q]],
                             _sel[b].at[pl.ds(q * _G, _G)], _sem_c[b])

    def wait_stage1(b):
        for q in range(_T // _G):
            pltpu.make_async_copy(codes_hbm.at[_ids[b].at[q]],
                                  _sel[b].at[pl.ds(q * _G, _G)],
                                  _sem_c[b]).wait()

    def chunk_body(g, b, prefetch_g):
        if prefetch_g is not None:
            issue_stage1(prefetch_g, 1 - b)
        wait_stage1(b)

        @pl.when(g >= 2)
        def _():
            pltpu.make_async_copy(_rows[b], out_slice(g - 2),
                                  _sem_o[b]).wait()

        gdescs = []
        for j in range(_T * _S // _G):
            for q in range(_G // 16):
                pair = j * (_G // 16) + q
                vals = plsc.load_gather(_sel[b], [2 * pair + tok_half, sub])
                _fidx[b][j, pl.ds(q * 16, 16)] = vals + sub_off
            gdescs.append(
                pltpu.async_copy(cb_sp.at[_fidx[b].at[j]],
                                 _rows[b].at[pl.ds(j * _G, _G)], sem_r))
        for dsc in gdescs:
            dsc.wait()
        pltpu.async_copy(_rows[b], out_slice(g), _sem_o[b])

    @pl.when(sid == 0)
    def _():
        pltpu.sync_copy(cb_hbm, cb_sp)
    plsc.subcore_barrier()

    issue_stage1(0, 0)

    def super_body(k, carry):
        g0 = 2 * k
        chunk_body(g0, 0, g0 + 1)
        chunk_body(g0 + 1, 1, g0 + 2)
        return carry

    lax.fori_loop(0, (_NCH - 1) // 2, super_body, 0)
    chunk_body(_NCH - 1, 0, None)

    pltpu.make_async_copy(rows1, out_slice(_NCH - 2), sem_o1).wait()
    pltpu.make_async_copy(rows0, out_slice(_NCH - 1), sem_o0).wait()


def kernel(input_ids, codebooks, codes):
    ids1d = input_ids.reshape(_NTOK).astype(jnp.int32)
    cb2d = codebooks.reshape(_S * _CBS, _D)
    out = _pq_lookup(ids1d, cb2d, codes)
    return out.reshape(_B, _L, _S * _D)
